# baseline (speedup 1.0000x reference)
import functools

import jax
import jax.numpy as jnp
from jax import lax
from jax.experimental import pallas as pl
from jax.experimental.pallas import tpu as pltpu
from jax.experimental.pallas import tpu_sc as plsc

_B = 16384
_D = 16
_PK = 128
_NC = 2
_NS = 16
_NW = _NC * _NS
_BPW = _B // _NW
_CH = 256
_NCHUNK = _BPW // _CH


def _sc_gather(user_pk, item_pk, cat_pk, uhi, ihi, chi):
    mesh = plsc.VectorSubcoreMesh(core_axis_name="c", subcore_axis_name="s")

    @functools.partial(
        pl.kernel,
        mesh=mesh,
        out_type=(
            jax.ShapeDtypeStruct((_B, _PK), jnp.float32),
            jax.ShapeDtypeStruct((_B, _PK), jnp.float32),
            jax.ShapeDtypeStruct((_B, _PK), jnp.float32),
        ),
        scratch_types=[
            pltpu.VMEM((_BPW,), jnp.int32),
            pltpu.VMEM((_BPW,), jnp.int32),
            pltpu.VMEM((_BPW,), jnp.int32),
            pltpu.VMEM((_CH, _PK), jnp.float32),
            pltpu.VMEM((_CH, _PK), jnp.float32),
            pltpu.VMEM((_CH, _PK), jnp.float32),
            pltpu.SemaphoreType.DMA,
            pltpu.SemaphoreType.DMA,
            pltpu.SemaphoreType.DMA,
        ],
    )
    def k(ut, it, ct, ui, ii, ci, uo, io, co,
          uiv, iiv, civ, urv, irv, crv, s0, s1, s2):
        wid = lax.axis_index("s") * _NC + lax.axis_index("c")
        base = wid * _BPW
        sl = pl.ds(base, _BPW)
        pltpu.sync_copy(ui.at[sl], uiv)
        pltpu.sync_copy(ii.at[sl], iiv)
        pltpu.sync_copy(ci.at[sl], civ)

        @pl.loop(0, _NCHUNK)
        def _(c):
            off = c * _CH
            isl = pl.ds(off, _CH)
            osl = pl.ds(base + off, _CH)
            c0 = pltpu.async_copy(ut.at[uiv.at[isl]], urv, s0)
            c1 = pltpu.async_copy(it.at[iiv.at[isl]], irv, s1)
            c2 = pltpu.async_copy(ct.at[civ.at[isl]], crv, s2)
            c0.wait()
            c1.wait()
            c2.wait()
            pltpu.sync_copy(urv, uo.at[osl])
            pltpu.sync_copy(irv, io.at[osl])
            pltpu.sync_copy(crv, co.at[osl])

    return k(user_pk, item_pk, cat_pk, uhi, ihi, chi)


_BLK = 2048


def _extract(packed, lo):
    out = jnp.zeros((packed.shape[0], _D), jnp.float32)
    for k in range(8):
        out = out + jnp.where(lo == k, packed[:, k * _D:(k + 1) * _D], 0.0)
    return out


def _dense_body(pu_ref, pi_ref, pc_ref, ulo_ref, ilo_ref, clo_ref,
                aid_ref, gid_ref, p_ref, agetab_ref, gentab_ref,
                uw1a_ref, uw1b_ref, uw1c_ref, ub1_ref, uw2_ref, ub2_ref,
                iw1a_ref, iw1b_ref, iw1c_ref, ib1_ref, iw2_ref, ib2_ref,
                o_ref):
    dot = lambda a, b: jnp.dot(a, b, preferred_element_type=jnp.float32)
    ue = _extract(pu_ref[...], ulo_ref[...])
    ie = _extract(pi_ref[...], ilo_ref[...])
    ce = _extract(pc_ref[...], clo_ref[...])

    n = pu_ref.shape[0]
    oh_a = (aid_ref[...] == lax.broadcasted_iota(jnp.int32, (n, 128), 1))
    ae = dot(oh_a.astype(jnp.float32), agetab_ref[...])
    oh_g = (gid_ref[...] == lax.broadcasted_iota(jnp.int32, (n, 8), 1))
    ge = dot(oh_g.astype(jnp.float32), gentab_ref[...])

    h = jnp.maximum(
        dot(ue, uw1a_ref[...]) + dot(ae, uw1b_ref[...])
        + dot(ge, uw1c_ref[...]) + ub1_ref[...], 0.0)
    uv = dot(h, uw2_ref[...]) + ub2_ref[...]

    h2 = jnp.maximum(
        dot(ie, iw1a_ref[...]) + dot(ce, iw1b_ref[...])
        + p_ref[...] * (1.0 / 1000.0) * iw1c_ref[...]
        + ib1_ref[...], 0.0)
    iv = dot(h2, iw2_ref[...]) + ib2_ref[...]

    num = jnp.sum(uv * iv, axis=1, keepdims=True)
    du = jnp.maximum(jnp.sqrt(jnp.sum(uv * uv, axis=1, keepdims=True)), 1e-12)
    di = jnp.maximum(jnp.sqrt(jnp.sum(iv * iv, axis=1, keepdims=True)), 1e-12)
    o_ref[...] = num / (du * di)


def _dense(pu, pi, pc, ulo, ilo, clo, aid, gid, price, agetab, gentab,
           uW1a, uW1b, uW1c, ub1, uW2, ub2, iW1a, iW1b, iW1c, ib1, iW2, ib2):
    full = lambda shape: pl.BlockSpec(shape, lambda i: (0, 0))
    rowpk = lambda: pl.BlockSpec((_BLK, _PK), lambda i: (i, 0))
    col1 = lambda: pl.BlockSpec((_BLK, 1), lambda i: (i, 0))
    return pl.pallas_call(
        _dense_body,
        grid=(_B // _BLK,),
        in_specs=[
            rowpk(), rowpk(), rowpk(),
            col1(), col1(), col1(), col1(), col1(), col1(),
            full((128, _D)),
            full((8, _D)),
            full((_D, _D)),
            full((_D, _D)),
            full((_D, _D)),
            full((1, _D)),
            full((_D, _D)),
            full((1, _D)),
            full((_D, _D)),
            full((_D, _D)),
            full((1, _D)),
            full((1, _D)),
            full((_D, _D)),
            full((1, _D)),
        ],
        out_specs=pl.BlockSpec((_BLK, 1), lambda i: (i, 0)),
        out_shape=jax.ShapeDtypeStruct((_B, 1), jnp.float32),
    )(pu, pi, pc, ulo, ilo, clo, aid, gid, price, agetab, gentab,
      uW1a, uW1b, uW1c, ub1, uW2, ub2, iW1a, iW1b, iW1c, ib1, iW2, ib2)


def kernel(user_id, age, gender, item_id, category, price,
           user_table, age_table, gender_table, item_table, cat_table,
           uW1, ub1, uW2, ub2, iW1, ib1, iW2, ib2):
    uid = user_id[:, 0].astype(jnp.int32)
    iid = item_id[:, 0].astype(jnp.int32)
    cid = category[:, 0].astype(jnp.int32)

    user_pk = user_table.reshape(-1, _PK)
    item_pk = item_table.reshape(-1, _PK)
    cat_pk = jnp.pad(cat_table, ((0, 0), (0, 8))).reshape(-1, _PK)

    pu, pi, pc = _sc_gather(user_pk, item_pk, cat_pk,
                            uid >> 3, iid >> 3, cid >> 3)

    agetab = jnp.pad(age_table, ((0, 28), (0, 8)))
    gentab = jnp.pad(gender_table, ((0, 5), (0, 12)))

    z8 = jnp.zeros((8, _D), jnp.float32)
    z12 = jnp.zeros((12, _D), jnp.float32)
    uW1a = uW1[0:16]
    uW1b = jnp.concatenate([uW1[16:24], z8], axis=0)
    uW1c = jnp.concatenate([uW1[24:28], z12], axis=0)
    iW1a = iW1[0:16]
    iW1b = jnp.concatenate([iW1[16:24], z8], axis=0)
    iW1c = iW1[24:25]

    logit = _dense(pu, pi, pc,
                   (user_id & 7).astype(jnp.int32),
                   (item_id & 7).astype(jnp.int32),
                   (category & 7).astype(jnp.int32),
                   age.astype(jnp.int32), gender.astype(jnp.int32), price,
                   agetab, gentab,
                   uW1a, uW1b, uW1c, ub1.reshape(1, _D), uW2, ub2.reshape(1, _D),
                   iW1a, iW1b, iW1c, ib1.reshape(1, _D), iW2, ib2.reshape(1, _D))
    return logit[:, 0]
